# Initial kernel scaffold; baseline (speedup 1.0000x reference)
import jax
import jax.numpy as jnp
from jax.experimental import pallas as pl


def kernel(x, gate_w, gate_b, w1, b1, w2, b2):
    raise NotImplementedError("write your pallas kernel here")


# speedup vs baseline: 2.3220x; 2.3220x over previous
import functools

import jax
import jax.numpy as jnp
from jax import lax
from jax.experimental import pallas as pl
from jax.experimental.pallas import tpu as pltpu
from jax.experimental.pallas import tpu_sc as plsc

BN = 4096
CD = 1024
HD = 4096
NE = 8
TK = 2
NA = BN * TK
T = 256
NT = NA // T + NE
PADDED = NT * T
EPAD = 128


def _gating_body(x_ref, gw_ref, gb_ref, ids_ref, wts_ref):
    xb = x_ref[...].astype(jnp.bfloat16)
    logits = jnp.dot(xb, gw_ref[...], preferred_element_type=jnp.float32)
    logits = logits + gb_ref[...]
    lane = lax.broadcasted_iota(jnp.int32, logits.shape, 1)
    big = jnp.int32(10 ** 9)
    m0 = jnp.max(logits, axis=1, keepdims=True)
    i0 = jnp.min(jnp.where(logits == m0, lane, big), axis=1, keepdims=True)
    l1 = jnp.where(lane == i0, jnp.float32(-1e30), logits)
    m1 = jnp.max(l1, axis=1, keepdims=True)
    i1 = jnp.min(jnp.where(l1 == m1, lane, big), axis=1, keepdims=True)
    d = jnp.exp(m1 - m0)
    w0 = 1.0 / (1.0 + d)
    ids_ref[:, 0:1] = i0
    ids_ref[:, 1:2] = i1
    wts_ref[:, 0:1] = w0
    wts_ref[:, 1:2] = d * w0


def _gating(xf, gwp, gbp):
    mt = 512
    return pl.pallas_call(
        _gating_body,
        grid=(BN // mt,),
        in_specs=[
            pl.BlockSpec((mt, CD), lambda m: (m, 0)),
            pl.BlockSpec((CD, EPAD), lambda m: (0, 0)),
            pl.BlockSpec((1, EPAD), lambda m: (0, 0)),
        ],
        out_specs=[
            pl.BlockSpec((mt, TK), lambda m: (m, 0)),
            pl.BlockSpec((mt, TK), lambda m: (m, 0)),
        ],
        out_shape=[
            jax.ShapeDtypeStruct((BN, TK), jnp.int32),
            jax.ShapeDtypeStruct((BN, TK), jnp.float32),
        ],
    )(xf, gwp, gbp)


NWK = 32
APW = NA // NWK
VPW = APW // 16
NV = NA // 16
RCH = 64
NCH = APW // RCH
TE_PAD = 64


def _route_sc_body(e_hbm, x_hbm, xs_hbm, inv_hbm, te_hbm,
                   ids_v, pos2_v, tok2_v, rows_v, te_v, sem_g, sem_s):
    cid = lax.axis_index("c")
    sid = lax.axis_index("s")
    wid = sid * 2 + cid
    base_a = wid * APW
    myv = wid * VPW
    pltpu.sync_copy(e_hbm, ids_v)

    zero = jnp.zeros((16,), jnp.int32)

    def hbody(i, accs):
        v = ids_v[pl.ds(i * 16, 16)]
        return tuple(accs[e] + jnp.where(v == e, 1, 0).astype(jnp.int32)
                     for e in range(NE))

    accs = lax.fori_loop(0, myv, hbody, (zero,) * NE)
    pre = [jnp.sum(a) for a in accs]
    accs = lax.fori_loop(myv, NV, hbody, accs)
    tot = [jnp.sum(a) for a in accs]

    bacc = jnp.int32(0)
    off, ends = [], []
    for e in range(NE):
        off.append(bacc + pre[e])
        bacc = bacc + ((tot[e] + (T - 1)) // T) * T
        ends.append(bacc)

    def pbody(j, offs):
        v = ids_v[pl.ds(base_a + j * 16, 16)]
        posv = zero
        new = []
        for e in range(NE):
            m = v == e
            ones = jnp.where(m, 1, 0).astype(jnp.int32)
            r = jnp.cumsum(ones)
            posv = jnp.where(m, offs[e] + r - 1, posv)
            new.append(offs[e] + jnp.sum(ones))
        row = j // NCH
        colbase = (j % NCH) * 16
        pos2_v[row, pl.ds(colbase, 16)] = posv
        tok2_v[row, pl.ds(colbase, 16)] = (
            (base_a + j * 16) + lax.iota(jnp.int32, 16)) // TK
        return tuple(new)

    lax.fori_loop(0, VPW, pbody, tuple(off))

    pltpu.sync_copy(pos2_v, inv_hbm.at[pl.ds(wid * NCH, NCH)])

    @pl.when(wid == 0)
    def _():
        for g in range(TE_PAD // 16):
            tv = lax.iota(jnp.int32, 16) + g * 16
            s = zero
            for e in range(NE):
                s = s + jnp.where(tv * T >= ends[e], 1, 0).astype(jnp.int32)
            te_v[pl.ds(g * 16, 16)] = jnp.minimum(s, NE - 1)
        pltpu.sync_copy(te_v, te_hbm)

    for c in range(NCH):
        pltpu.async_copy(x_hbm.at[tok2_v.at[c]], rows_v, sem_g).wait()
        pltpu.async_copy(rows_v, xs_hbm.at[pos2_v.at[c]], sem_s).wait()


def _route(e_flat, x_i32):
    mesh = plsc.VectorSubcoreMesh(core_axis_name="c", subcore_axis_name="s", num_cores=2, num_subcores=16)
    f = functools.partial(
        pl.kernel,
        out_type=[
            jax.ShapeDtypeStruct((PADDED, CD // 2), jnp.int32),
            jax.ShapeDtypeStruct((NWK * NCH, RCH), jnp.int32),
            jax.ShapeDtypeStruct((TE_PAD,), jnp.int32),
        ],
        mesh=mesh,
        compiler_params=pltpu.CompilerParams(needs_layout_passes=False),
        scratch_types=[
            pltpu.VMEM((NA,), jnp.int32),
            pltpu.VMEM((NCH, RCH), jnp.int32),
            pltpu.VMEM((NCH, RCH), jnp.int32),
            pltpu.VMEM((RCH, CD // 2), jnp.int32),
            pltpu.VMEM((TE_PAD,), jnp.int32),
            pltpu.SemaphoreType.DMA,
            pltpu.SemaphoreType.DMA,
        ],
    )
    x_sorted, inv2, te = f(_route_sc_body)(e_flat, x_i32)
    return x_sorted, inv2.reshape(NA), te[:NT]


def _gelu_exact(h):
    return 0.5 * h * (1.0 + lax.erf(h * 0.7071067811865476))


def _mlp_body(te_ref, x_ref, w1_ref, b1_ref, w2_ref, b2_ref, y_ref):
    xb = x_ref[...]
    h = jnp.dot(xb, w1_ref[0], preferred_element_type=jnp.float32)
    h = _gelu_exact(h + b1_ref[0])
    y = jnp.dot(h.astype(jnp.bfloat16), w2_ref[0],
                preferred_element_type=jnp.float32)
    y_ref[...] = y + b2_ref[0]


def _expert_mlp(tile_e, x_sorted, w1b, b1r, w2b, b2r):
    grid_spec = pltpu.PrefetchScalarGridSpec(
        num_scalar_prefetch=1,
        grid=(NT,),
        in_specs=[
            pl.BlockSpec((T, CD), lambda t, te: (t, 0)),
            pl.BlockSpec((1, CD, HD), lambda t, te: (te[t], 0, 0)),
            pl.BlockSpec((1, 1, HD), lambda t, te: (te[t], 0, 0)),
            pl.BlockSpec((1, HD, CD), lambda t, te: (te[t], 0, 0)),
            pl.BlockSpec((1, 1, CD), lambda t, te: (te[t], 0, 0)),
        ],
        out_specs=pl.BlockSpec((T, CD), lambda t, te: (t, 0)),
    )
    return pl.pallas_call(
        _mlp_body,
        grid_spec=grid_spec,
        out_shape=jax.ShapeDtypeStruct((PADDED, CD), jnp.float32),
    )(tile_e, x_sorted, w1b, b1r, w2b, b2r)


TPW = BN // NWK
DCH = 32
DNCH = TPW // DCH


def _combine_sc_body(y_hbm, inv_hbm, wts_hbm, out_hbm,
                     inv_v, wts_v, rows_v, out_v, sem):
    cid = lax.axis_index("c")
    sid = lax.axis_index("s")
    wid = sid * 2 + cid
    tok0 = wid * TPW
    pltpu.sync_copy(inv_hbm.at[pl.ds(tok0 * TK, TPW * TK)], inv_v)
    pltpu.sync_copy(wts_hbm.at[pl.ds(tok0 * TK, TPW * TK)],
                    wts_v.at[pl.ds(0, TPW * TK)])

    def chunk(c, _):
        idxs = inv_v.at[pl.ds(c * DCH * TK, DCH * TK)]
        pltpu.async_copy(y_hbm.at[idxs], rows_v, sem).wait()

        def tok(t, _):
            wv = wts_v[pl.ds((c * DCH + t) * TK, 16)]
            w0 = wv[0]
            w1 = wv[1]
            for k in range(CD // 16):
                s = pl.ds(k * 16, 16)
                out_v[t, s] = w0 * rows_v[2 * t, s] + w1 * rows_v[2 * t + 1, s]
            return 0

        lax.fori_loop(0, DCH, tok, 0)
        pltpu.sync_copy(out_v, out_hbm.at[pl.ds(tok0 + c * DCH, DCH)])
        return 0

    lax.fori_loop(0, DNCH, chunk, 0)


def _combine(y_sorted, inv, wts_flat):
    mesh = plsc.VectorSubcoreMesh(core_axis_name="c", subcore_axis_name="s", num_cores=2, num_subcores=16)
    f = functools.partial(
        pl.kernel,
        out_type=jax.ShapeDtypeStruct((BN, CD), jnp.float32),
        mesh=mesh,
        compiler_params=pltpu.CompilerParams(needs_layout_passes=False),
        scratch_types=[
            pltpu.VMEM((TPW * TK,), jnp.int32),
            pltpu.VMEM((TPW * TK + 16,), jnp.float32),
            pltpu.VMEM((DCH * TK, CD), jnp.float32),
            pltpu.VMEM((DCH, CD), jnp.float32),
            pltpu.SemaphoreType.DMA,
        ],
    )
    return f(_combine_sc_body)(y_sorted, inv, wts_flat)


def kernel(x, gate_w, gate_b, w1, b1, w2, b2):
    Bd, Nd, Cd = x.shape
    xf = x.reshape(BN, CD)

    gwp = jnp.zeros((CD, EPAD), jnp.bfloat16)
    gwp = gwp.at[:, :NE].set(gate_w.astype(jnp.bfloat16))
    gbp = jnp.full((1, EPAD), -1e30, jnp.float32)
    gbp = gbp.at[0, :NE].set(gate_b)

    ids, wts = _gating(xf, gwp, gbp)
    e_flat = ids.reshape(NA)
    wts_flat = wts.reshape(NA)

    x_bf = xf.astype(jnp.bfloat16)
    x_i32 = lax.bitcast_convert_type(x_bf.reshape(BN, CD // 2, 2), jnp.int32)
    x_sorted_i32, inv, tile_e = _route(e_flat, x_i32)
    x_sorted = lax.bitcast_convert_type(
        x_sorted_i32, jnp.bfloat16).reshape(PADDED, CD)

    y_sorted = _expert_mlp(tile_e, x_sorted,
                           w1.astype(jnp.bfloat16), b1.reshape(NE, 1, HD),
                           w2.astype(jnp.bfloat16), b2.reshape(NE, 1, CD))

    out = _combine(y_sorted, inv, wts_flat)
    return out.reshape(Bd, Nd, Cd)


# speedup vs baseline: 3.1427x; 1.3534x over previous
import functools

import jax
import jax.numpy as jnp
from jax import lax
from jax.experimental import pallas as pl
from jax.experimental.pallas import tpu as pltpu
from jax.experimental.pallas import tpu_sc as plsc

BN = 4096
CD = 1024
HD = 4096
NE = 8
TK = 2
NA = BN * TK
T = 256
NT = NA // T + NE
PADDED = NT * T
EPAD = 128


def _gating_body(x_ref, gw_ref, gb_ref, ids_ref, wts_ref):
    xb = x_ref[...].astype(jnp.bfloat16)
    gw = gw_ref[...].astype(jnp.bfloat16)
    logits = jnp.dot(xb, gw, preferred_element_type=jnp.float32)
    logits = logits + gb_ref[...]
    lane = lax.broadcasted_iota(jnp.int32, logits.shape, 1)
    big = jnp.int32(10 ** 9)
    m0 = jnp.max(logits, axis=1, keepdims=True)
    i0 = jnp.min(jnp.where(logits == m0, lane, big), axis=1, keepdims=True)
    l1 = jnp.where(lane == i0, jnp.float32(-1e30), logits)
    m1 = jnp.max(l1, axis=1, keepdims=True)
    i1 = jnp.min(jnp.where(l1 == m1, lane, big), axis=1, keepdims=True)
    d = jnp.exp(m1 - m0)
    w0 = 1.0 / (1.0 + d)
    ids_ref[:, 0:1] = i0
    ids_ref[:, 1:2] = i1
    wts_ref[:, 0:1] = w0
    wts_ref[:, 1:2] = d * w0


def _gating(xf, gw, gb):
    mt = 512
    return pl.pallas_call(
        _gating_body,
        grid=(BN // mt,),
        in_specs=[
            pl.BlockSpec((mt, CD), lambda m: (m, 0)),
            pl.BlockSpec((CD, NE), lambda m: (0, 0)),
            pl.BlockSpec((1, NE), lambda m: (0, 0)),
        ],
        out_specs=[
            pl.BlockSpec((mt, TK), lambda m: (m, 0)),
            pl.BlockSpec((mt, TK), lambda m: (m, 0)),
        ],
        out_shape=[
            jax.ShapeDtypeStruct((BN, TK), jnp.int32),
            jax.ShapeDtypeStruct((BN, TK), jnp.float32),
        ],
    )(xf, gw, gb)


NWK = 32
APW = NA // NWK
VPW = APW // 16
NV = NA // 16
RCH = 64
NCH = APW // RCH
TE_PAD = 64


def _route_sc_body(e_hbm, x_hbm, xs_hbm, inv_hbm, te_hbm,
                   ids_v, pos2_v, tok2_v, rows_v, te_v, sem_g, sem_s):
    cid = lax.axis_index("c")
    sid = lax.axis_index("s")
    wid = sid * 2 + cid
    base_a = wid * APW
    myv = wid * VPW
    pltpu.sync_copy(e_hbm, ids_v)

    zero = jnp.zeros((16,), jnp.int32)

    def hbody(i, accs):
        v = ids_v[pl.ds(i * 16, 16)]
        return tuple(accs[e] + jnp.where(v == e, 1, 0).astype(jnp.int32)
                     for e in range(NE))

    accs = lax.fori_loop(0, myv, hbody, (zero,) * NE)
    pre = [jnp.sum(a) for a in accs]
    accs = lax.fori_loop(myv, NV, hbody, accs)
    tot = [jnp.sum(a) for a in accs]

    bacc = jnp.int32(0)
    off, ends = [], []
    for e in range(NE):
        off.append(bacc + pre[e])
        bacc = bacc + ((tot[e] + (T - 1)) // T) * T
        ends.append(bacc)

    def pbody(j, offs):
        v = ids_v[pl.ds(base_a + j * 16, 16)]
        posv = zero
        new = []
        for e in range(NE):
            m = v == e
            ones = jnp.where(m, 1, 0).astype(jnp.int32)
            r = jnp.cumsum(ones)
            posv = jnp.where(m, offs[e] + r - 1, posv)
            new.append(offs[e] + jnp.sum(ones))
        row = j // NCH
        colbase = (j % NCH) * 16
        pos2_v[row, pl.ds(colbase, 16)] = posv
        tok2_v[row, pl.ds(colbase, 16)] = (
            (base_a + j * 16) + lax.iota(jnp.int32, 16)) // TK
        return tuple(new)

    lax.fori_loop(0, VPW, pbody, tuple(off))

    pltpu.sync_copy(pos2_v, inv_hbm.at[pl.ds(wid * NCH, NCH)])

    @pl.when(wid == 0)
    def _():
        for g in range(TE_PAD // 16):
            tv = lax.iota(jnp.int32, 16) + g * 16
            s = zero
            for e in range(NE):
                s = s + jnp.where(tv * T >= ends[e], 1, 0).astype(jnp.int32)
            te_v[pl.ds(g * 16, 16)] = jnp.minimum(s, NE - 1)
        pltpu.sync_copy(te_v, te_hbm)

    for c in range(NCH):
        pltpu.async_copy(x_hbm.at[tok2_v.at[c]], rows_v, sem_g).wait()
        pltpu.async_copy(rows_v, xs_hbm.at[pos2_v.at[c]], sem_s).wait()


def _route(e_flat, xf):
    mesh = plsc.VectorSubcoreMesh(core_axis_name="c", subcore_axis_name="s", num_cores=2, num_subcores=16)
    f = functools.partial(
        pl.kernel,
        out_type=[
            jax.ShapeDtypeStruct((PADDED, CD), jnp.float32),
            jax.ShapeDtypeStruct((NWK * NCH, RCH), jnp.int32),
            jax.ShapeDtypeStruct((TE_PAD,), jnp.int32),
        ],
        mesh=mesh,
        compiler_params=pltpu.CompilerParams(needs_layout_passes=False),
        scratch_types=[
            pltpu.VMEM((NA,), jnp.int32),
            pltpu.VMEM((NCH, RCH), jnp.int32),
            pltpu.VMEM((NCH, RCH), jnp.int32),
            pltpu.VMEM((RCH, CD), jnp.float32),
            pltpu.VMEM((TE_PAD,), jnp.int32),
            pltpu.SemaphoreType.DMA,
            pltpu.SemaphoreType.DMA,
        ],
    )
    x_sorted, inv2, te = f(_route_sc_body)(e_flat, xf)
    return x_sorted, inv2.reshape(NA), te[:NT]


def _gelu_exact(h):
    return 0.5 * h * (1.0 + lax.erf(h * 0.7071067811865476))


HB = 2
HS = HD // HB


def _mlp_body(te_ref, x_ref, w1_ref, b1_ref, w2_ref, b2_ref, y_ref):
    hstep = pl.program_id(1)
    xb = x_ref[...].astype(jnp.bfloat16)
    w1 = w1_ref[0].astype(jnp.bfloat16)
    h = jnp.dot(xb, w1, preferred_element_type=jnp.float32)
    h = _gelu_exact(h + b1_ref[0])
    w2 = w2_ref[0].astype(jnp.bfloat16)
    y = jnp.dot(h.astype(jnp.bfloat16), w2, preferred_element_type=jnp.float32)

    @pl.when(hstep == 0)
    def _():
        y_ref[...] = y

    @pl.when(hstep == HB - 1)
    def _():
        y_ref[...] += y + b2_ref[0]


def _expert_mlp(tile_e, x_sorted, w1b, b1r, w2b, b2r):
    grid_spec = pltpu.PrefetchScalarGridSpec(
        num_scalar_prefetch=1,
        grid=(NT, HB),
        in_specs=[
            pl.BlockSpec((T, CD), lambda t, h, te: (t, 0)),
            pl.BlockSpec((1, CD, HS), lambda t, h, te: (te[t], 0, h)),
            pl.BlockSpec((1, 1, HS), lambda t, h, te: (te[t], 0, h)),
            pl.BlockSpec((1, HS, CD), lambda t, h, te: (te[t], h, 0)),
            pl.BlockSpec((1, 1, CD), lambda t, h, te: (te[t], 0, 0)),
        ],
        out_specs=pl.BlockSpec((T, CD), lambda t, h, te: (t, 0)),
    )
    return pl.pallas_call(
        _mlp_body,
        grid_spec=grid_spec,
        out_shape=jax.ShapeDtypeStruct((PADDED, CD), jnp.float32),
    )(tile_e, x_sorted, w1b, b1r, w2b, b2r)


TPW = BN // NWK
DCH = 32
DNCH = TPW // DCH


def _combine_sc_body(y_hbm, inv_hbm, wts_hbm, out_hbm,
                     inv_v, wts_v, rows_v, out_v, sem):
    cid = lax.axis_index("c")
    sid = lax.axis_index("s")
    wid = sid * 2 + cid
    tok0 = wid * TPW
    pltpu.sync_copy(inv_hbm.at[pl.ds(tok0 * TK, TPW * TK)], inv_v)
    pltpu.sync_copy(wts_hbm.at[pl.ds(tok0 * TK, TPW * TK)],
                    wts_v.at[pl.ds(0, TPW * TK)])

    def chunk(c, _):
        idxs = inv_v.at[pl.ds(c * DCH * TK, DCH * TK)]
        pltpu.async_copy(y_hbm.at[idxs], rows_v, sem).wait()

        def tok(t, _):
            wv = wts_v[pl.ds((c * DCH + t) * TK, 16)]
            w0 = wv[0]
            w1 = wv[1]
            for k in range(CD // 16):
                s = pl.ds(k * 16, 16)
                out_v[t, s] = w0 * rows_v[2 * t, s] + w1 * rows_v[2 * t + 1, s]
            return 0

        lax.fori_loop(0, DCH, tok, 0)
        pltpu.sync_copy(out_v, out_hbm.at[pl.ds(tok0 + c * DCH, DCH)])
        return 0

    lax.fori_loop(0, DNCH, chunk, 0)


def _combine(y_sorted, inv, wts_flat):
    mesh = plsc.VectorSubcoreMesh(core_axis_name="c", subcore_axis_name="s", num_cores=2, num_subcores=16)
    f = functools.partial(
        pl.kernel,
        out_type=jax.ShapeDtypeStruct((BN, CD), jnp.float32),
        mesh=mesh,
        compiler_params=pltpu.CompilerParams(needs_layout_passes=False),
        scratch_types=[
            pltpu.VMEM((TPW * TK,), jnp.int32),
            pltpu.VMEM((TPW * TK + 16,), jnp.float32),
            pltpu.VMEM((DCH * TK, CD), jnp.float32),
            pltpu.VMEM((DCH, CD), jnp.float32),
            pltpu.SemaphoreType.DMA,
        ],
    )
    return f(_combine_sc_body)(y_sorted, inv, wts_flat)


def kernel(x, gate_w, gate_b, w1, b1, w2, b2):
    Bd, Nd, Cd = x.shape
    xf = x.reshape(BN, CD)

    ids, wts = _gating(xf, gate_w, gate_b.reshape(1, NE))
    e_flat = ids.reshape(NA)
    wts_flat = wts.reshape(NA)

    x_sorted, inv, tile_e = _route(e_flat, xf)

    y_sorted = _expert_mlp(tile_e, x_sorted,
                           w1, b1.reshape(NE, 1, HD),
                           w2, b2.reshape(NE, 1, CD))

    out = _combine(y_sorted, inv, wts_flat)
    return out.reshape(Bd, Nd, Cd)


# speedup vs baseline: 3.9116x; 1.2447x over previous
import functools

import jax
import jax.numpy as jnp
from jax import lax
from jax.experimental import pallas as pl
from jax.experimental.pallas import tpu as pltpu
from jax.experimental.pallas import tpu_sc as plsc

BN = 4096
CD = 1024
HD = 4096
NE = 8
TK = 2
NA = BN * TK
T = 256
NT = NA // T + NE
PADDED = NT * T
EPAD = 128


def _gating_body(x_ref, gw_ref, gb_ref, ids_ref, wts_ref):
    xb = x_ref[...].astype(jnp.bfloat16)
    gw = gw_ref[...].astype(jnp.bfloat16)
    logits = jnp.dot(xb, gw, preferred_element_type=jnp.float32)
    logits = logits + gb_ref[...]
    lane = lax.broadcasted_iota(jnp.int32, logits.shape, 1)
    big = jnp.int32(10 ** 9)
    m0 = jnp.max(logits, axis=1, keepdims=True)
    i0 = jnp.min(jnp.where(logits == m0, lane, big), axis=1, keepdims=True)
    l1 = jnp.where(lane == i0, jnp.float32(-1e30), logits)
    m1 = jnp.max(l1, axis=1, keepdims=True)
    i1 = jnp.min(jnp.where(l1 == m1, lane, big), axis=1, keepdims=True)
    d = jnp.exp(m1 - m0)
    w0 = 1.0 / (1.0 + d)
    ids_ref[:, 0:1] = i0
    ids_ref[:, 1:2] = i1
    wts_ref[:, 0:1] = w0
    wts_ref[:, 1:2] = d * w0


def _gating(xf, gw, gb):
    mt = 512
    return pl.pallas_call(
        _gating_body,
        grid=(BN // mt,),
        in_specs=[
            pl.BlockSpec((mt, CD), lambda m: (m, 0)),
            pl.BlockSpec((CD, NE), lambda m: (0, 0)),
            pl.BlockSpec((1, NE), lambda m: (0, 0)),
        ],
        out_specs=[
            pl.BlockSpec((mt, TK), lambda m: (m, 0)),
            pl.BlockSpec((mt, TK), lambda m: (m, 0)),
        ],
        out_shape=[
            jax.ShapeDtypeStruct((BN, TK), jnp.int32),
            jax.ShapeDtypeStruct((BN, TK), jnp.float32),
        ],
    )(xf, gw, gb)


NWK = 32
APW = NA // NWK
VPW = APW // 16
NV = NA // 16
RCH = 64
NCH = APW // RCH
TE_PAD = 64


def _route_sc_body(e_hbm, x_hbm, xs_hbm, inv_hbm, te_hbm,
                   ids_v, pos2_v, tok2_v, rows_v, te_v, sem_g, sem_s):
    cid = lax.axis_index("c")
    sid = lax.axis_index("s")
    wid = sid * 2 + cid
    base_a = wid * APW
    myv = wid * VPW
    pltpu.sync_copy(e_hbm, ids_v)

    zero = jnp.zeros((16,), jnp.int32)

    def hbody(i, accs):
        v = ids_v[pl.ds(i * 16, 16)]
        return tuple(accs[e] + jnp.where(v == e, 1, 0).astype(jnp.int32)
                     for e in range(NE))

    accs = lax.fori_loop(0, myv, hbody, (zero,) * NE)
    pre = [jnp.sum(a) for a in accs]
    accs = lax.fori_loop(myv, NV, hbody, accs)
    tot = [jnp.sum(a) for a in accs]

    bacc = jnp.int32(0)
    off, ends = [], []
    for e in range(NE):
        off.append(bacc + pre[e])
        bacc = bacc + ((tot[e] + (T - 1)) // T) * T
        ends.append(bacc)

    def pbody(j, offs):
        v = ids_v[pl.ds(base_a + j * 16, 16)]
        posv = zero
        new = []
        for e in range(NE):
            m = v == e
            ones = jnp.where(m, 1, 0).astype(jnp.int32)
            r = jnp.cumsum(ones)
            posv = jnp.where(m, offs[e] + r - 1, posv)
            new.append(offs[e] + jnp.sum(ones))
        row = j // NCH
        colbase = (j % NCH) * 16
        pos2_v[row, pl.ds(colbase, 16)] = posv
        tok2_v[row, pl.ds(colbase, 16)] = (
            (base_a + j * 16) + lax.iota(jnp.int32, 16)) // TK
        return tuple(new)

    lax.fori_loop(0, VPW, pbody, tuple(off))

    pltpu.sync_copy(pos2_v, inv_hbm.at[pl.ds(wid * NCH, NCH)])

    @pl.when(wid == 0)
    def _():
        for g in range(TE_PAD // 16):
            tv = lax.iota(jnp.int32, 16) + g * 16
            s = zero
            for e in range(NE):
                s = s + jnp.where(tv * T >= ends[e], 1, 0).astype(jnp.int32)
            te_v[pl.ds(g * 16, 16)] = jnp.minimum(s, NE - 1)
        pltpu.sync_copy(te_v, te_hbm)

    for c in range(NCH):
        pltpu.async_copy(x_hbm.at[tok2_v.at[c]], rows_v, sem_g).wait()
        pltpu.async_copy(rows_v, xs_hbm.at[pos2_v.at[c]], sem_s).wait()


def _route(e_flat, xf):
    mesh = plsc.VectorSubcoreMesh(core_axis_name="c", subcore_axis_name="s", num_cores=2, num_subcores=16)
    f = functools.partial(
        pl.kernel,
        out_type=[
            jax.ShapeDtypeStruct((PADDED, CD), jnp.float32),
            jax.ShapeDtypeStruct((NWK * NCH, RCH), jnp.int32),
            jax.ShapeDtypeStruct((TE_PAD,), jnp.int32),
        ],
        mesh=mesh,
        compiler_params=pltpu.CompilerParams(needs_layout_passes=False),
        scratch_types=[
            pltpu.VMEM((NA,), jnp.int32),
            pltpu.VMEM((NCH, RCH), jnp.int32),
            pltpu.VMEM((NCH, RCH), jnp.int32),
            pltpu.VMEM((RCH, CD), jnp.float32),
            pltpu.VMEM((TE_PAD,), jnp.int32),
            pltpu.SemaphoreType.DMA,
            pltpu.SemaphoreType.DMA,
        ],
    )
    x_sorted, inv2, te = f(_route_sc_body)(e_flat, xf)
    return x_sorted, inv2.reshape(NA), te[:NT]


def _gelu_exact(h):
    return 0.5 * h * (1.0 + lax.erf(h * 0.7071067811865476))


HB = 2
HS = HD // HB


def _mlp_body(te_ref, x_ref, w1_ref, b1_ref, w2_ref, b2_ref, y_ref):
    hstep = pl.program_id(1)
    xb = x_ref[...].astype(jnp.bfloat16)
    w1 = w1_ref[0].astype(jnp.bfloat16)
    h = jnp.dot(xb, w1, preferred_element_type=jnp.float32)
    h = _gelu_exact(h + b1_ref[0])
    w2 = w2_ref[0].astype(jnp.bfloat16)
    y = jnp.dot(h.astype(jnp.bfloat16), w2, preferred_element_type=jnp.float32)

    @pl.when(hstep == 0)
    def _():
        y_ref[...] = y

    @pl.when(hstep == HB - 1)
    def _():
        y_ref[...] += y + b2_ref[0]


def _expert_mlp(tile_e, x_sorted, w1b, b1r, w2b, b2r):
    grid_spec = pltpu.PrefetchScalarGridSpec(
        num_scalar_prefetch=1,
        grid=(NT, HB),
        in_specs=[
            pl.BlockSpec((T, CD), lambda t, h, te: (t, 0)),
            pl.BlockSpec((1, CD, HS), lambda t, h, te: (te[t], 0, h)),
            pl.BlockSpec((1, 1, HS), lambda t, h, te: (te[t], 0, h)),
            pl.BlockSpec((1, HS, CD), lambda t, h, te: (te[t], h, 0)),
            pl.BlockSpec((1, 1, CD), lambda t, h, te: (te[t], 0, 0)),
        ],
        out_specs=pl.BlockSpec((T, CD), lambda t, h, te: (t, 0)),
    )
    return pl.pallas_call(
        _mlp_body,
        grid_spec=grid_spec,
        out_shape=jax.ShapeDtypeStruct((PADDED, CD), jnp.float32),
    )(tile_e, x_sorted, w1b, b1r, w2b, b2r)


TPW = BN // NWK
DCH = 16
DNCH = TPW // DCH


def _combine_sc_body(y_hbm, inv_hbm, wts_hbm, out_hbm,
                     inv_v, wts_v, rows0, rows1, out0, out1,
                     sg0, sg1, sw0, sw1):
    cid = lax.axis_index("c")
    sid = lax.axis_index("s")
    wid = sid * 2 + cid
    tok0 = wid * TPW
    pltpu.sync_copy(inv_hbm.at[pl.ds(tok0 * TK, TPW * TK)], inv_v)
    pltpu.sync_copy(wts_hbm.at[pl.ds(tok0 * TK, TPW * TK)],
                    wts_v.at[pl.ds(0, TPW * TK)])

    rows = (rows0, rows1)
    outs = (out0, out1)
    sg = (sg0, sg1)
    sw = (sw0, sw1)
    gh = [None] * DNCH
    wh = [None] * DNCH

    gh[0] = pltpu.async_copy(
        y_hbm.at[inv_v.at[pl.ds(0, DCH * TK)]], rows[0], sg[0])
    for c in range(DNCH):
        b = c % 2
        if c + 1 < DNCH:
            gh[c + 1] = pltpu.async_copy(
                y_hbm.at[inv_v.at[pl.ds((c + 1) * DCH * TK, DCH * TK)]],
                rows[1 - b], sg[1 - b])
        gh[c].wait()
        if c >= 2:
            wh[c - 2].wait()

        def tok(t, _, c=c, b=b):
            wv = wts_v[pl.ds((c * DCH + t) * TK, 16)]
            w0 = wv[0]
            w1 = wv[1]
            for k in range(CD // 16):
                s = pl.ds(k * 16, 16)
                outs[b][t, s] = (w0 * rows[b][2 * t, s]
                                 + w1 * rows[b][2 * t + 1, s])
            return 0

        lax.fori_loop(0, DCH, tok, 0)
        wh[c] = pltpu.async_copy(
            outs[b], out_hbm.at[pl.ds(tok0 + c * DCH, DCH)], sw[b])
    wh[DNCH - 2].wait()
    wh[DNCH - 1].wait()


def _combine(y_sorted, inv, wts_flat):
    mesh = plsc.VectorSubcoreMesh(core_axis_name="c", subcore_axis_name="s", num_cores=2, num_subcores=16)
    f = functools.partial(
        pl.kernel,
        out_type=jax.ShapeDtypeStruct((BN, CD), jnp.float32),
        mesh=mesh,
        compiler_params=pltpu.CompilerParams(needs_layout_passes=False),
        scratch_types=[
            pltpu.VMEM((TPW * TK,), jnp.int32),
            pltpu.VMEM((TPW * TK + 16,), jnp.float32),
            pltpu.VMEM((DCH * TK, CD), jnp.float32),
            pltpu.VMEM((DCH * TK, CD), jnp.float32),
            pltpu.VMEM((DCH, CD), jnp.float32),
            pltpu.VMEM((DCH, CD), jnp.float32),
            pltpu.SemaphoreType.DMA,
            pltpu.SemaphoreType.DMA,
            pltpu.SemaphoreType.DMA,
            pltpu.SemaphoreType.DMA,
        ],
    )
    return f(_combine_sc_body)(y_sorted, inv, wts_flat)


def kernel(x, gate_w, gate_b, w1, b1, w2, b2):
    Bd, Nd, Cd = x.shape
    xf = x.reshape(BN, CD)

    ids, wts = _gating(xf, gate_w, gate_b.reshape(1, NE))
    e_flat = ids.reshape(NA)
    wts_flat = wts.reshape(NA)

    x_sorted, inv, tile_e = _route(e_flat, xf)

    y_sorted = _expert_mlp(tile_e, x_sorted,
                           w1, b1.reshape(NE, 1, HD),
                           w2, b2.reshape(NE, 1, CD))

    out = _combine(y_sorted, inv, wts_flat)
    return out.reshape(Bd, Nd, Cd)
